# baseline (speedup 1.0000x reference)
import functools

import jax
import jax.numpy as jnp
from jax.experimental import pallas as pl
from jax.experimental.pallas import tpu as pltpu

DEPTH = 4
DIM = 1024
HEADS = 8
DH = DIM // HEADS
FF = 4 * DIM
BUCKET = 64
S = 2048
NB = S // BUCKET
NROT = NB // 2
C = S // BUCKET
RB = 256
NRB = S // RB


def _ref_ln(x, s, b):
    m = jnp.mean(x, axis=-1, keepdims=True)
    v = jnp.var(x, axis=-1, keepdims=True)
    return (x - m) / jnp.sqrt(v + 1e-5) * s + b


def _hash_body(qk_ref, rotblk_ref, keys_ref):
    i = pl.program_id(0)
    rot = jnp.dot(qk_ref[...], rotblk_ref[...],
                  preferred_element_type=jnp.float32)
    lane = jax.lax.broadcasted_iota(jnp.int32, (RB, NROT), 1)
    cols = []
    for hd in range(HEADS):
        r = rot[:, hd * NROT:(hd + 1) * NROT]
        mx_p = jnp.max(r, axis=-1, keepdims=True)
        mx_n = jnp.max(-r, axis=-1, keepdims=True)
        idx_p = jnp.min(jnp.where(r == mx_p, lane, NB), axis=-1, keepdims=True)
        idx_n = jnp.min(jnp.where(-r == mx_n, lane, NB), axis=-1, keepdims=True)
        cols.append(jnp.where(mx_p >= mx_n, idx_p, NROT + idx_n))
    buckets = jnp.concatenate(cols, axis=1)
    row = i * RB + jax.lax.broadcasted_iota(jnp.int32, (RB, HEADS), 0)
    keys_ref[0] = buckets * S + row


def _hash_call(qk, rotblk):
    return pl.pallas_call(
        _hash_body,
        grid=(NRB,),
        in_specs=[
            pl.BlockSpec((RB, DIM), lambda i: (i, 0)),
            pl.BlockSpec((DIM, HEADS * NROT), lambda i: (0, 0)),
        ],
        out_specs=pl.BlockSpec((1, RB, HEADS), lambda i: (i, 0, 0)),
        out_shape=jax.ShapeDtypeStruct((NRB, RB, HEADS), jnp.int32),
    )(qk, rotblk)


def _rank_body(kr_ref, kc_ref, rank_ref):
    kc = kc_ref[0]
    kr = kr_ref[0]
    for jb in range(S // DH):
        krb = jax.lax.slice(kr, (0, jb * DH), (1, (jb + 1) * DH))
        lt = (kc < krb).astype(jnp.int32)
        rank_ref[0, pl.ds(jb, 1), :] = jnp.sum(lt, axis=0, keepdims=True)


def _rank_call(keys_r, keys_c):
    return pl.pallas_call(
        _rank_body,
        grid=(HEADS,),
        in_specs=[
            pl.BlockSpec((1, 1, S), lambda h: (h, 0, 0)),
            pl.BlockSpec((1, S, 1), lambda h: (h, 0, 0)),
        ],
        out_specs=pl.BlockSpec((1, S // DH, DH), lambda h: (h, 0, 0)),
        out_shape=jax.ShapeDtypeStruct((HEADS, S // DH, DH), jnp.int32),
    )(keys_r, keys_c)


def _dots_body(qk_ref, nk_ref, v_ref, rank_ref, dots_ref, sv_ref,
               sqk_ref, sk_ref):
    rank_row = rank_ref[0]
    for sc in range(NRB):
        iota_col = sc * RB + jax.lax.broadcasted_iota(jnp.int32, (RB, 1), 0)
        g = (rank_row == iota_col).astype(jnp.float32)
        sqk_ref[pl.ds(sc * RB, RB), :] = jnp.dot(
            g, qk_ref[...], preferred_element_type=jnp.float32,
            precision=jax.lax.Precision.HIGHEST)
        sk_ref[pl.ds(sc * RB, RB), :] = jnp.dot(
            g, nk_ref[...], preferred_element_type=jnp.float32,
            precision=jax.lax.Precision.HIGHEST)
        sv_ref[pl.ds(sc * RB, RB), :] = jnp.dot(
            g, v_ref[...], preferred_element_type=jnp.float32,
            precision=jax.lax.Precision.HIGHEST)

    def chunk(c, _):
        pc = jax.lax.rem(c + C - 1, C)
        q = sqk_ref[pl.ds(c * BUCKET, BUCKET), :]
        kcat = jnp.concatenate(
            [sk_ref[pl.ds(c * BUCKET, BUCKET), :],
             sk_ref[pl.ds(pc * BUCKET, BUCKET), :]], axis=0)
        dots = jax.lax.dot_general(
            q, kcat, (((1,), (1,)), ((), ())),
            preferred_element_type=jnp.float32) / jnp.sqrt(jnp.float32(DH))
        ri = jax.lax.broadcasted_iota(jnp.int32, (BUCKET, 2 * BUCKET), 0)
        ci = jax.lax.broadcasted_iota(jnp.int32, (BUCKET, 2 * BUCKET), 1)
        dots_ref[0, pl.ds(c * BUCKET, BUCKET), :] = jnp.where(
            ri == ci, -5e4, dots)
        return 0

    jax.lax.fori_loop(0, C, chunk, 0)


def _dots_call(qk, nk, v, rank_r):
    return pl.pallas_call(
        _dots_body,
        grid=(HEADS,),
        in_specs=[
            pl.BlockSpec((S, DH), lambda h: (0, h)),
            pl.BlockSpec((S, DH), lambda h: (0, h)),
            pl.BlockSpec((S, DH), lambda h: (0, h)),
            pl.BlockSpec((1, 1, S), lambda h: (h, 0, 0)),
        ],
        out_specs=[
            pl.BlockSpec((1, S, 2 * BUCKET), lambda h: (h, 0, 0)),
            pl.BlockSpec((S, DH), lambda h: (0, h)),
        ],
        out_shape=[
            jax.ShapeDtypeStruct((HEADS, S, 2 * BUCKET), jnp.float32),
            jax.ShapeDtypeStruct((S, DIM), jnp.float32),
        ],
        scratch_shapes=[
            pltpu.VMEM((S, DH), jnp.float32),
            pltpu.VMEM((S, DH), jnp.float32),
        ],
    )(qk, nk, v, rank_r)


def _bo_body(attn_ref, sv_ref, rank_ref, o_ref, so_ref):
    rank_row = rank_ref[0]

    def chunk(c, _):
        pc = jax.lax.rem(c + C - 1, C)
        a = attn_ref[0, pl.ds(c * BUCKET, BUCKET), :]
        vcat = jnp.concatenate(
            [sv_ref[pl.ds(c * BUCKET, BUCKET), :],
             sv_ref[pl.ds(pc * BUCKET, BUCKET), :]], axis=0)
        so_ref[pl.ds(c * BUCKET, BUCKET), :] = jnp.dot(
            a, vcat, preferred_element_type=jnp.float32)
        return 0

    jax.lax.fori_loop(0, C, chunk, 0)

    for sc in range(NRB):
        iota_col = sc * RB + jax.lax.broadcasted_iota(jnp.int32, (RB, 1), 0)
        g = (rank_row == iota_col).astype(jnp.float32)
        part = jax.lax.dot_general(
            g, so_ref[pl.ds(sc * RB, RB), :], (((0,), (0,)), ((), ())),
            preferred_element_type=jnp.float32,
            precision=jax.lax.Precision.HIGHEST)
        if sc == 0:
            o_ref[...] = part
        else:
            o_ref[...] += part


def _bo_call(attn, sv, rank_r):
    return pl.pallas_call(
        _bo_body,
        grid=(HEADS,),
        in_specs=[
            pl.BlockSpec((1, S, 2 * BUCKET), lambda h: (h, 0, 0)),
            pl.BlockSpec((S, DH), lambda h: (0, h)),
            pl.BlockSpec((1, 1, S), lambda h: (h, 0, 0)),
        ],
        out_specs=pl.BlockSpec((S, DH), lambda h: (0, h)),
        out_shape=jax.ShapeDtypeStruct((S, DIM), jnp.float32),
        scratch_shapes=[
            pltpu.VMEM((S, DH), jnp.float32),
        ],
    )(attn, sv, rank_r)


def kernel(x, mask, Wqk, Wv, Wo, ln1_s, ln1_b, W1, b1, W2, b2, ln2_s, ln2_b):
    del mask
    rots = jax.random.normal(jax.random.key(42),
                             (DEPTH, HEADS, DH, NROT), dtype=jnp.float32)
    rrow = jnp.arange(DIM)[:, None] // DH
    rcol = jnp.arange(HEADS * NROT)[None, :] // NROT
    blkmask = (rrow == rcol).astype(jnp.float32)
    h = x
    for l in range(DEPTH):
        rotblk = jnp.tile(rots[l].reshape(DIM, NROT), (1, HEADS)) * blkmask
        xn = _ref_ln(h, ln1_s[l], ln1_b[l])
        qk = (xn @ Wqk[l])[0]
        v = (xn @ Wv[l])[0]
        keys = _hash_call(qk, rotblk)
        keys_r = keys.reshape(S, HEADS).T.reshape(HEADS, 1, S)
        keys_c = keys_r.reshape(HEADS, S, 1)
        rank = _rank_call(keys_r, keys_c)
        rank_r = rank.reshape(HEADS, 1, S)
        qkr = qk.reshape(S, HEADS, DH)
        nk = (qkr / (jnp.linalg.norm(qkr, axis=-1, keepdims=True) + 1e-6)
              ).reshape(S, DIM)
        dots, sv = _dots_call(qk, nk, v, rank_r)
        attn = jax.nn.softmax(
            dots.reshape(1, HEADS, C, BUCKET, 2 * BUCKET), axis=-1
        ).reshape(HEADS, S, 2 * BUCKET)
        o = _bo_call(attn, sv, rank_r)
        h = h + (o[None] @ Wo[l])
        ffh = _ref_ln(h, ln2_s[l], ln2_b[l])
        h = h + (jax.nn.gelu(ffh @ W1[l] + b1[l]) @ W2[l] + b2[l])
    return h
